# Initial kernel scaffold; baseline (speedup 1.0000x reference)
import jax
import jax.numpy as jnp
from jax.experimental import pallas as pl


def kernel(x, edge_index, W1, b1, W2, b2, Wo, bo):
    raise NotImplementedError("write your pallas kernel here")


# speedup vs baseline: 17.5405x; 17.5405x over previous
import functools

import jax
import jax.numpy as jnp
from jax import lax
from jax.experimental import pallas as pl
from jax.experimental.pallas import tpu as pltpu
from jax.experimental.pallas import tpu_sc as plsc

NN = 10000
FD = 128
H2_ = 64
NE = 320000
NC_, NS_ = 2, 16
NW_ = NC_ * NS_
EPW = NE // NW_
CH = 128
NFULL = EPW // CH
REM = EPW - NFULL * CH
NP_ = 10240
DPT = NP_ // NS_
RPT = 632
RPT_LAST = NN - 15 * RPT

_mesh = plsc.VectorSubcoreMesh(core_axis_name="c", subcore_axis_name="s")


@functools.partial(
    pl.kernel,
    out_type=jax.ShapeDtypeStruct((NC_, NP_), jnp.float32),
    mesh=_mesh,
    scratch_types=[
        pltpu.VMEM((CH,), jnp.int32),
        pltpu.VMEM((REM,), jnp.int32),
        pltpu.VMEM((CH,), jnp.float32),
        pltpu.VMEM((REM,), jnp.float32),
        pltpu.VMEM((DPT,), jnp.float32),
        pltpu.VMEM_SHARED((NP_,), jnp.float32),
    ],
)
def _deg_sc(dst_hbm, out_hbm, idx_v, idxr_v, ones_v, onesr_v, zbuf_v, deg_sh):
    c = lax.axis_index("c")
    s = lax.axis_index("s")
    wid = s * NC_ + c
    base = pl.multiple_of(wid * EPW, 8)
    dof = pl.multiple_of(s * DPT, 8)
    for i in range(DPT // 16):
        zbuf_v[pl.ds(i * 16, 16)] = jnp.zeros((16,), jnp.float32)
    for i in range(CH // 16):
        ones_v[pl.ds(i * 16, 16)] = jnp.full((16,), 1.0, jnp.float32)
    onesr_v[pl.ds(0, 16)] = jnp.full((16,), 1.0, jnp.float32)
    pltpu.sync_copy(zbuf_v, deg_sh.at[pl.ds(dof, DPT)])
    plsc.subcore_barrier()

    def body(j, carry):
        off = pl.multiple_of(base + j * CH, 8)
        pltpu.sync_copy(dst_hbm.at[pl.ds(off, CH)], idx_v)
        pltpu.sync_copy(ones_v, deg_sh.at[idx_v], add=True)
        return carry

    lax.fori_loop(0, NFULL, body, 0)
    offr = pl.multiple_of(base + NFULL * CH, 8)
    pltpu.sync_copy(dst_hbm.at[pl.ds(offr, REM)], idxr_v)
    pltpu.sync_copy(onesr_v, deg_sh.at[idxr_v], add=True)
    plsc.subcore_barrier()
    pltpu.sync_copy(deg_sh.at[pl.ds(dof, DPT)],
                    out_hbm.at[c, pl.ds(dof, DPT)])


def _make_agg(F):
    @functools.partial(
        pl.kernel,
        out_type=jax.ShapeDtypeStruct((NC_, NN, F), jnp.float32),
        mesh=_mesh,
        scratch_types=[
            pltpu.VMEM((CH,), jnp.int32),
            pltpu.VMEM((CH,), jnp.int32),
            pltpu.VMEM((CH, F), jnp.float32),
            pltpu.VMEM((REM,), jnp.int32),
            pltpu.VMEM((REM,), jnp.int32),
            pltpu.VMEM((REM, F), jnp.float32),
            pltpu.SemaphoreType.DMA,
            pltpu.VMEM_SHARED((NN, F), jnp.float32),
        ],
        compiler_params=pltpu.CompilerParams(use_tc_tiling_on_sc=False),
    )
    def agg(g_hbm, z_hbm, src_hbm, dst_hbm, out_hbm,
            src_v, dst_v, rows_v, srcr_v, dstr_v, rowsr_v, sem, acc_sh):
        c = lax.axis_index("c")
        s = lax.axis_index("s")
        wid = s * NC_ + c
        base = pl.multiple_of(wid * EPW, 8)
        row0 = pl.multiple_of(s * RPT, 8)

        def _rows_copy(from_g):
            src_ref = g_hbm if from_g else z_hbm

            @pl.when(s < 15)
            def _():
                pltpu.sync_copy(src_ref.at[pl.ds(row0, RPT), :],
                                acc_sh.at[pl.ds(row0, RPT), :])

            @pl.when(s == 15)
            def _():
                pltpu.sync_copy(src_ref.at[pl.ds(15 * RPT, RPT_LAST), :],
                                acc_sh.at[pl.ds(15 * RPT, RPT_LAST), :])

        @pl.when(c == 0)
        def _():
            _rows_copy(True)

        @pl.when(c == 1)
        def _():
            _rows_copy(False)

        plsc.subcore_barrier()

        def body(j, carry):
            off = pl.multiple_of(base + j * CH, 8)
            pltpu.sync_copy(src_hbm.at[pl.ds(off, CH)], src_v)
            pltpu.sync_copy(dst_hbm.at[pl.ds(off, CH)], dst_v)
            pltpu.async_copy(g_hbm.at[src_v], rows_v, sem).wait()
            pltpu.sync_copy(rows_v, acc_sh.at[dst_v], add=True)
            return carry

        lax.fori_loop(0, NFULL, body, 0)
        offr = pl.multiple_of(base + NFULL * CH, 8)
        pltpu.sync_copy(src_hbm.at[pl.ds(offr, REM)], srcr_v)
        pltpu.sync_copy(dst_hbm.at[pl.ds(offr, REM)], dstr_v)
        pltpu.async_copy(g_hbm.at[srcr_v], rowsr_v, sem).wait()
        pltpu.sync_copy(rowsr_v, acc_sh.at[dstr_v], add=True)

        plsc.subcore_barrier()

        @pl.when(s < 15)
        def _():
            pltpu.sync_copy(acc_sh.at[pl.ds(row0, RPT), :],
                            out_hbm.at[c, pl.ds(row0, RPT), :])

        @pl.when(s == 15)
        def _():
            pltpu.sync_copy(acc_sh.at[pl.ds(15 * RPT, RPT_LAST), :],
                            out_hbm.at[c, pl.ds(15 * RPT, RPT_LAST), :])

    return agg


_agg128 = _make_agg(FD)
_agg64 = _make_agg(H2_)


_BR = 2000
_GRID = NN // _BR


def _t1_body(degT_ref, x_ref, w1_ref, g_ref, dis_ref):
    deg = jnp.sum(degT_ref[...], axis=1, keepdims=True) + 1.0
    dis = lax.rsqrt(deg)
    g_ref[...] = jnp.dot(x_ref[...], w1_ref[...],
                         preferred_element_type=jnp.float32) * dis
    dis_ref[...] = dis


def _t1(degT, x, W1):
    return pl.pallas_call(
        _t1_body,
        grid=(_GRID,),
        in_specs=[
            pl.BlockSpec((_BR, 2), lambda j: (j, 0)),
            pl.BlockSpec((_BR, FD), lambda j: (j, 0)),
            pl.BlockSpec((FD, FD), lambda j: (0, 0)),
        ],
        out_specs=[
            pl.BlockSpec((_BR, FD), lambda j: (j, 0)),
            pl.BlockSpec((_BR, 1), lambda j: (j, 0)),
        ],
        out_shape=[
            jax.ShapeDtypeStruct((NN, FD), jnp.float32),
            jax.ShapeDtypeStruct((NN, 1), jnp.float32),
        ],
    )(degT, x, W1)


def _t2_body(p_ref, dis_ref, b1_ref, w2_ref, g2_ref):
    dis = dis_ref[...]
    h1 = jnp.maximum(dis * (p_ref[0] + p_ref[1]) + b1_ref[...], 0.0)
    g2_ref[...] = jnp.dot(h1, w2_ref[...],
                          preferred_element_type=jnp.float32) * dis


def _t2(p, dis, b1, W2):
    return pl.pallas_call(
        _t2_body,
        grid=(_GRID,),
        in_specs=[
            pl.BlockSpec((NC_, _BR, FD), lambda j: (0, j, 0)),
            pl.BlockSpec((_BR, 1), lambda j: (j, 0)),
            pl.BlockSpec((1, FD), lambda j: (0, 0)),
            pl.BlockSpec((FD, H2_), lambda j: (0, 0)),
        ],
        out_specs=pl.BlockSpec((_BR, H2_), lambda j: (j, 0)),
        out_shape=jax.ShapeDtypeStruct((NN, H2_), jnp.float32),
    )(p, dis, b1, W2)


def _t3_body(q_ref, dis_ref, b2_ref, w0_ref, w1_ref, bo_ref, o0_ref, o1_ref):
    dis = dis_ref[...]
    h2 = jnp.maximum(dis * (q_ref[0] + q_ref[1]) + b2_ref[...], 0.0)
    l0 = jnp.sum(h2 * w0_ref[...], axis=1, keepdims=True) + bo_ref[:, 0:1]
    l1 = jnp.sum(h2 * w1_ref[...], axis=1, keepdims=True) + bo_ref[:, 1:2]
    m = jnp.maximum(l0, l1)
    lse = m + jnp.log(jnp.exp(l0 - m) + jnp.exp(l1 - m))
    o0_ref[...] = l0 - lse
    o1_ref[...] = l1 - lse


def _t3(q, dis, b2, w0, w1, bo2):
    return pl.pallas_call(
        _t3_body,
        grid=(_GRID,),
        in_specs=[
            pl.BlockSpec((NC_, _BR, H2_), lambda j: (0, j, 0)),
            pl.BlockSpec((_BR, 1), lambda j: (j, 0)),
            pl.BlockSpec((1, H2_), lambda j: (0, 0)),
            pl.BlockSpec((1, H2_), lambda j: (0, 0)),
            pl.BlockSpec((1, H2_), lambda j: (0, 0)),
            pl.BlockSpec((1, 2), lambda j: (0, 0)),
        ],
        out_specs=[
            pl.BlockSpec((_BR, 1), lambda j: (j, 0)),
            pl.BlockSpec((_BR, 1), lambda j: (j, 0)),
        ],
        out_shape=[
            jax.ShapeDtypeStruct((NN, 1), jnp.float32),
            jax.ShapeDtypeStruct((NN, 1), jnp.float32),
        ],
    )(q, dis, b2, w0, w1, bo2)


def kernel(x, edge_index, W1, b1, W2, b2, Wo, bo):
    src = edge_index[0]
    dst = edge_index[1]
    degp = _deg_sc(dst)
    degT = jnp.transpose(degp[:, :NN])
    g1, dis = _t1(degT, x, W1)
    z128 = jnp.zeros((NN, FD), jnp.float32)
    p = _agg128(g1, z128, src, dst)
    g2 = _t2(p, dis, b1.reshape(1, FD), W2)
    z64 = jnp.zeros((NN, H2_), jnp.float32)
    q = _agg64(g2, z64, src, dst)
    o0, o1 = _t3(q, dis, b2.reshape(1, H2_),
                 Wo[:, 0].reshape(1, H2_), Wo[:, 1].reshape(1, H2_),
                 bo.reshape(1, 2))
    return jnp.concatenate([o0, o1], axis=1)
